# baseline (speedup 1.0000x reference)
import functools

import jax
import jax.numpy as jnp
from jax import lax
from jax.experimental import pallas as pl
from jax.experimental.pallas import tpu as pltpu
from jax.experimental.pallas import tpu_sc as plsc

N = 10000
D = 128
E = 320000

NC = 2
NS = 16
NW = NC * NS
CH = 128
CPT = 81
EPT = CPT * CH
EPAD = NW * EPT
NROWS = 10112
RPT = NROWS // NS
RFULL = (RPT // CH) * CH
RREM = RPT - RFULL

_f32 = jnp.float32


@functools.lru_cache(maxsize=None)
def _sc_mesh():
    return plsc.VectorSubcoreMesh(core_axis_name="c", subcore_axis_name="s",
                                  num_cores=NC, num_subcores=NS)


def _deg_body(dstr_hbm, out_hbm, acc_sh, dst_v, buf_v):
    cid = lax.axis_index("c")
    sid = lax.axis_index("s")
    wid = cid * NS + sid
    row0 = sid * RPT

    def fill(val):
        def body(r, carry):
            buf_v[r, :] = jnp.full((16,), val, _f32)
            return carry
        lax.fori_loop(0, CH, body, 0)

    fill(0.0)
    for k in range(RPT // CH):
        pltpu.sync_copy(buf_v, acc_sh.at[pl.ds(row0 + k * CH, CH)])
    if RREM:
        pltpu.sync_copy(buf_v.at[pl.ds(0, RREM)],
                        acc_sh.at[pl.ds(row0 + RFULL, RREM)])
    fill(1.0)

    pltpu.sync_copy(dstr_hbm.at[wid], dst_v)
    plsc.subcore_barrier()

    def chunk(j, carry):
        pltpu.sync_copy(buf_v, acc_sh.at[dst_v.at[j]], add=True)
        return carry
    lax.fori_loop(0, CPT, chunk, 0)

    plsc.subcore_barrier()

    for k in range(RPT // CH):
        pltpu.sync_copy(acc_sh.at[pl.ds(row0 + k * CH, CH)], buf_v)
        pltpu.sync_copy(buf_v, out_hbm.at[cid, pl.ds(row0 + k * CH, CH)])
    if RREM:
        pltpu.sync_copy(acc_sh.at[pl.ds(row0 + RFULL, RREM)],
                        buf_v.at[pl.ds(0, RREM)])
        pltpu.sync_copy(buf_v.at[pl.ds(0, RREM)],
                        out_hbm.at[cid, pl.ds(row0 + RFULL, RREM)])


@functools.lru_cache(maxsize=None)
def _deg_call():
    return pl.kernel(
        _deg_body,
        out_type=jax.ShapeDtypeStruct((NC, NROWS, 16), _f32),
        mesh=_sc_mesh(),
        scratch_types=[
            pltpu.VMEM_SHARED((NROWS, 16), _f32),
            pltpu.VMEM((CPT, CH), jnp.int32),
            pltpu.VMEM((CH, 16), _f32),
        ],
    )


NBUF = 3


def _agg_body(hs_hbm, er_hbm, out_hbm, acc_sh, *bufs_and_sems):
    rows = bufs_and_sems[:NBUF]
    idx = bufs_and_sems[NBUF:2 * NBUF]
    semg = bufs_and_sems[2 * NBUF:3 * NBUF]
    sems = bufs_and_sems[3 * NBUF:4 * NBUF]
    cid = lax.axis_index("c")
    sid = lax.axis_index("s")
    wid = cid * NS + sid
    row0 = sid * RPT

    def zfill(i, carry):
        r = i // 8
        c = (i % 8) * 16
        rows[0][r, pl.ds(c, 16)] = jnp.zeros((16,), _f32)
        return carry
    lax.fori_loop(0, CH * 8, zfill, 0)
    for k in range(RPT // CH):
        pltpu.sync_copy(rows[0], acc_sh.at[pl.ds(row0 + k * CH, CH)])
    if RREM:
        pltpu.sync_copy(rows[0].at[pl.ds(0, RREM)],
                        acc_sh.at[pl.ds(row0 + RFULL, RREM)])
    plsc.subcore_barrier()

    def load_idx(b, j):
        pltpu.sync_copy(er_hbm.at[wid, j], idx[b])

    def gather(b):
        return pltpu.make_async_copy(hs_hbm.at[idx[b].at[0]], rows[b],
                                     semg[b])

    def scatter(b):
        return pltpu.make_async_copy(rows[b], acc_sh.at[idx[b].at[1]],
                                     sems[b])

    for b in range(NBUF):
        load_idx(b, b)
        gather(b).start()

    def group(g, carry):
        for b in range(NBUF):
            gather(b).wait()
            pltpu.async_copy(rows[b], acc_sh.at[idx[b].at[1]], sems[b],
                             add=True)
        for b in range(NBUF):
            j = g * NBUF + b
            scatter(b).wait()

            @pl.when(j + NBUF < CPT)
            def _():
                load_idx(b, j + NBUF)
                gather(b).start()
        return carry
    lax.fori_loop(0, CPT // NBUF, group, 0)

    plsc.subcore_barrier()

    for k in range(RPT // CH):
        pltpu.sync_copy(acc_sh.at[pl.ds(row0 + k * CH, CH)], rows[0])
        pltpu.sync_copy(rows[0], out_hbm.at[cid, pl.ds(row0 + k * CH, CH)])
    if RREM:
        pltpu.sync_copy(acc_sh.at[pl.ds(row0 + RFULL, RREM)],
                        rows[0].at[pl.ds(0, RREM)])
        pltpu.sync_copy(rows[0].at[pl.ds(0, RREM)],
                        out_hbm.at[cid, pl.ds(row0 + RFULL, RREM)])


@functools.lru_cache(maxsize=None)
def _agg_call():
    return pl.kernel(
        _agg_body,
        out_type=jax.ShapeDtypeStruct((NC, NROWS, D), _f32),
        mesh=_sc_mesh(),
        scratch_types=(
            [pltpu.VMEM_SHARED((NROWS, D), _f32)]
            + [pltpu.VMEM((CH, D), _f32) for _ in range(NBUF)]
            + [pltpu.VMEM((2, CH), jnp.int32) for _ in range(NBUF)]
            + [pltpu.SemaphoreType.DMA for _ in range(2 * NBUF)]
        ),
    )


_B = 1000


def _pre_body(d_ref, x_ref, w_ref, dinv_ref, hs_ref):
    deg = d_ref[0, :, 0:1] + d_ref[1, :, 0:1] + 1.0
    dinv = lax.rsqrt(deg)
    dinv_ref[...] = jnp.broadcast_to(dinv, (_B, D))
    hs_ref[...] = jnp.dot(x_ref[...] * dinv, w_ref[...],
                          preferred_element_type=_f32)


_pre_call = pl.pallas_call(
    _pre_body,
    grid=(N // _B,),
    in_specs=[
        pl.BlockSpec((NC, _B, 16), lambda i: (0, i, 0)),
        pl.BlockSpec((_B, D), lambda i: (i, 0)),
        pl.BlockSpec((D, D), lambda i: (0, 0)),
    ],
    out_specs=[
        pl.BlockSpec((_B, D), lambda i: (i, 0)),
        pl.BlockSpec((_B, D), lambda i: (i, 0)),
    ],
    out_shape=[
        jax.ShapeDtypeStruct((N, D), _f32),
        jax.ShapeDtypeStruct((N, D), _f32),
    ],
)


def _mid_body(acc_ref, hs_ref, dinv_ref, b_ref, w_ref, out_ref):
    t = (acc_ref[0] + acc_ref[1] + hs_ref[...]) * dinv_ref[...] + b_ref[...]
    t = jnp.maximum(t, 0.0)
    out_ref[...] = jnp.dot(t * dinv_ref[...], w_ref[...],
                           preferred_element_type=_f32)


_mid_call = pl.pallas_call(
    _mid_body,
    grid=(N // _B,),
    in_specs=[
        pl.BlockSpec((NC, _B, D), lambda i: (0, i, 0)),
        pl.BlockSpec((_B, D), lambda i: (i, 0)),
        pl.BlockSpec((_B, D), lambda i: (i, 0)),
        pl.BlockSpec((1, D), lambda i: (0, 0)),
        pl.BlockSpec((D, D), lambda i: (0, 0)),
    ],
    out_specs=pl.BlockSpec((_B, D), lambda i: (i, 0)),
    out_shape=jax.ShapeDtypeStruct((N, D), _f32),
)


def _fin_body(acc_ref, hs_ref, dinv_ref, b_ref, out_ref):
    out_ref[...] = ((acc_ref[0] + acc_ref[1] + hs_ref[...])
                    * dinv_ref[...] + b_ref[...])


_fin_call = pl.pallas_call(
    _fin_body,
    grid=(N // _B,),
    in_specs=[
        pl.BlockSpec((NC, _B, D), lambda i: (0, i, 0)),
        pl.BlockSpec((_B, D), lambda i: (i, 0)),
        pl.BlockSpec((_B, D), lambda i: (i, 0)),
        pl.BlockSpec((1, D), lambda i: (0, 0)),
    ],
    out_specs=pl.BlockSpec((_B, D), lambda i: (i, 0)),
    out_shape=jax.ShapeDtypeStruct((N, D), _f32),
)


def kernel(x, edge_index, W1, b1, W2, b2, W3, b3):
    ei = edge_index.astype(jnp.int32)
    pad = EPAD - E
    src = jnp.concatenate([ei[0], jnp.zeros((pad,), jnp.int32)])
    dst = jnp.concatenate([ei[1], jnp.full((pad,), N, jnp.int32)])
    srcr = src.reshape(NW, CPT, CH)
    dstr = dst.reshape(NW, CPT, CH)
    er = jnp.stack([srcr, dstr], axis=2)

    degs = _deg_call()(dstr)
    dinv, hs = _pre_call(degs[:, :N], x, W1)

    agg = _agg_call()
    acc = agg(hs, er)
    hs = _mid_call(acc[:, :N], hs, dinv, b1.reshape(1, D), W2)
    acc = agg(hs, er)
    hs = _mid_call(acc[:, :N], hs, dinv, b2.reshape(1, D), W3)
    acc = agg(hs, er)
    return _fin_call(acc[:, :N], hs, dinv, b3.reshape(1, D))


# speedup vs baseline: 2.9004x; 2.9004x over previous
import functools

import jax
import jax.numpy as jnp
from jax import lax
from jax.experimental import pallas as pl
from jax.experimental.pallas import tpu as pltpu
from jax.experimental.pallas import tpu_sc as plsc

N = 10000
D = 128
E = 320000

NC = 2
NS = 16
NW = NC * NS
CH = 128
NBUF = 3
CPT = 81
EPT = CPT * CH
EPAD = NW * EPT
NROWS = 10112
RPT = NROWS // NS
RFULL = (RPT // CH) * CH
RREM = RPT - RFULL
NR_DEG = 10240
RPT_DEG = NR_DEG // NS
CPT_DEG = 80
EPAD_DEG = NW * CPT_DEG * CH

_f32 = jnp.float32


@functools.lru_cache(maxsize=None)
def _sc_mesh():
    return plsc.VectorSubcoreMesh(core_axis_name="c", subcore_axis_name="s",
                                  num_cores=NC, num_subcores=NS)


def _deg_body(dstr_hbm, out_hbm, acc_sh, dst_v, buf_v):
    cid = lax.axis_index("c")
    sid = lax.axis_index("s")
    wid = cid * NS + sid
    row0 = sid * RPT_DEG

    def fill(val):
        def body(r, carry):
            buf_v[r, :] = jnp.full((16,), val, _f32)
            return carry
        lax.fori_loop(0, CH, body, 0)

    fill(0.0)
    for k in range(RPT_DEG // CH):
        pltpu.sync_copy(buf_v, acc_sh.at[pl.ds(row0 + k * CH, CH)])
    fill(1.0)

    pltpu.sync_copy(dstr_hbm.at[wid], dst_v)
    plsc.subcore_barrier()

    def chunk(j, carry):
        pltpu.sync_copy(buf_v, acc_sh.at[dst_v.at[j]], add=True)
        return carry
    lax.fori_loop(0, CPT_DEG, chunk, 0)

    plsc.subcore_barrier()

    for k in range(RPT_DEG // CH):
        pltpu.sync_copy(acc_sh.at[pl.ds(row0 + k * CH, CH)], buf_v)
        pltpu.sync_copy(buf_v, out_hbm.at[cid, pl.ds(row0 + k * CH, CH)])


@functools.lru_cache(maxsize=None)
def _deg_call():
    return pl.kernel(
        _deg_body,
        out_type=jax.ShapeDtypeStruct((NC, NR_DEG, 16), _f32),
        mesh=_sc_mesh(),
        scratch_types=[
            pltpu.VMEM_SHARED((NR_DEG, 16), _f32),
            pltpu.VMEM((CPT_DEG, CH), jnp.int32),
            pltpu.VMEM((CH, 16), _f32),
        ],
    )


def _agg_body(hs_hbm, srcr_hbm, dstr_hbm, out_hbm, msg_hbm, acc_sh,
              *bufs_and_sems):
    rows = bufs_and_sems[:NBUF]
    idx = bufs_and_sems[NBUF:2 * NBUF]
    semg = bufs_and_sems[2 * NBUF:3 * NBUF]
    sems = bufs_and_sems[3 * NBUF:4 * NBUF]
    cid = lax.axis_index("c")
    sid = lax.axis_index("s")
    wid = cid * NS + sid
    row0 = sid * RPT
    ebase = wid * EPT

    pltpu.sync_copy(hs_hbm.at[pl.ds(row0, RPT)], acc_sh.at[pl.ds(row0, RPT)])
    plsc.subcore_barrier()

    def gather(b):
        return pltpu.make_async_copy(acc_sh.at[idx[b].at[0]], rows[b],
                                     semg[b])

    def msg_w(b, j):
        return pltpu.make_async_copy(rows[b],
                                     msg_hbm.at[pl.ds(ebase + j * CH, CH)],
                                     sems[b])

    for b in range(NBUF):
        pltpu.sync_copy(srcr_hbm.at[wid, b], idx[b])
        gather(b).start()

    def group_a(g, carry):
        for b in range(NBUF):
            j = g * NBUF + b
            gather(b).wait()
            msg_w(b, j).start()
        for b in range(NBUF):
            j = g * NBUF + b
            msg_w(b, j).wait()

            @pl.when(j + NBUF < CPT)
            def _():
                pltpu.sync_copy(srcr_hbm.at[wid, j + NBUF], idx[b])
                gather(b).start()
        return carry
    lax.fori_loop(0, CPT // NBUF, group_a, 0)

    plsc.subcore_barrier()

    def zfill(i, carry):
        r = i // 8
        c = (i % 8) * 16
        rows[0][r, pl.ds(c, 16)] = jnp.zeros((16,), _f32)
        return carry
    lax.fori_loop(0, CH * 8, zfill, 0)
    for k in range(RPT // CH):
        pltpu.sync_copy(rows[0], acc_sh.at[pl.ds(row0 + k * CH, CH)])
    if RREM:
        pltpu.sync_copy(rows[0].at[pl.ds(0, RREM)],
                        acc_sh.at[pl.ds(row0 + RFULL, RREM)])
    plsc.subcore_barrier()

    def msg_r(b, j):
        return pltpu.make_async_copy(msg_hbm.at[pl.ds(ebase + j * CH, CH)],
                                     rows[b], semg[b])

    def scat(b):
        return pltpu.make_async_copy(rows[b], acc_sh.at[idx[b].at[0]],
                                     sems[b])

    for b in range(NBUF):
        pltpu.sync_copy(dstr_hbm.at[wid, b], idx[b])
        msg_r(b, b).start()

    def group_b(g, carry):
        for b in range(NBUF):
            j = g * NBUF + b
            msg_r(b, j).wait()
            pltpu.async_copy(rows[b], acc_sh.at[idx[b].at[0]], sems[b],
                             add=True)
        for b in range(NBUF):
            j = g * NBUF + b
            scat(b).wait()

            @pl.when(j + NBUF < CPT)
            def _():
                pltpu.sync_copy(dstr_hbm.at[wid, j + NBUF], idx[b])
                msg_r(b, j + NBUF).start()
        return carry
    lax.fori_loop(0, CPT // NBUF, group_b, 0)

    plsc.subcore_barrier()

    for k in range(RPT // CH):
        pltpu.sync_copy(acc_sh.at[pl.ds(row0 + k * CH, CH)], rows[0])
        pltpu.sync_copy(rows[0], out_hbm.at[cid, pl.ds(row0 + k * CH, CH)])
    if RREM:
        pltpu.sync_copy(acc_sh.at[pl.ds(row0 + RFULL, RREM)],
                        rows[0].at[pl.ds(0, RREM)])
        pltpu.sync_copy(rows[0].at[pl.ds(0, RREM)],
                        out_hbm.at[cid, pl.ds(row0 + RFULL, RREM)])


@functools.lru_cache(maxsize=None)
def _agg_call():
    return pl.kernel(
        _agg_body,
        out_type=[jax.ShapeDtypeStruct((NC, NROWS, D), _f32),
                  jax.ShapeDtypeStruct((EPAD, D), _f32)],
        mesh=_sc_mesh(),
        scratch_types=(
            [pltpu.VMEM_SHARED((NROWS, D), _f32)]
            + [pltpu.VMEM((CH, D), _f32) for _ in range(NBUF)]
            + [pltpu.VMEM((1, CH), jnp.int32) for _ in range(NBUF)]
            + [pltpu.SemaphoreType.DMA for _ in range(2 * NBUF)]
        ),
    )


_B = 1000


def _pre_body(d_ref, x_ref, w_ref, dinv_ref, hs_ref):
    deg = d_ref[0, :, 0:1] + d_ref[1, :, 0:1] + 1.0
    dinv = lax.rsqrt(deg)
    dinv_ref[...] = jnp.broadcast_to(dinv, (_B, D))
    hs_ref[...] = jnp.dot(x_ref[...] * dinv, w_ref[...],
                          preferred_element_type=_f32)


_pre_call = pl.pallas_call(
    _pre_body,
    grid=(N // _B,),
    in_specs=[
        pl.BlockSpec((NC, _B, 16), lambda i: (0, i, 0)),
        pl.BlockSpec((_B, D), lambda i: (i, 0)),
        pl.BlockSpec((D, D), lambda i: (0, 0)),
    ],
    out_specs=[
        pl.BlockSpec((_B, D), lambda i: (i, 0)),
        pl.BlockSpec((_B, D), lambda i: (i, 0)),
    ],
    out_shape=[
        jax.ShapeDtypeStruct((N, D), _f32),
        jax.ShapeDtypeStruct((NROWS, D), _f32),
    ],
)


def _mid_body(acc_ref, hs_ref, dinv_ref, b_ref, w_ref, out_ref):
    t = (acc_ref[0] + acc_ref[1] + hs_ref[...]) * dinv_ref[...] + b_ref[...]
    t = jnp.maximum(t, 0.0)
    out_ref[...] = jnp.dot(t * dinv_ref[...], w_ref[...],
                           preferred_element_type=_f32)


_mid_call = pl.pallas_call(
    _mid_body,
    grid=(N // _B,),
    in_specs=[
        pl.BlockSpec((NC, _B, D), lambda i: (0, i, 0)),
        pl.BlockSpec((_B, D), lambda i: (i, 0)),
        pl.BlockSpec((_B, D), lambda i: (i, 0)),
        pl.BlockSpec((1, D), lambda i: (0, 0)),
        pl.BlockSpec((D, D), lambda i: (0, 0)),
    ],
    out_specs=pl.BlockSpec((_B, D), lambda i: (i, 0)),
    out_shape=jax.ShapeDtypeStruct((NROWS, D), _f32),
)


def _fin_body(acc_ref, hs_ref, dinv_ref, b_ref, out_ref):
    out_ref[...] = ((acc_ref[0] + acc_ref[1] + hs_ref[...])
                    * dinv_ref[...] + b_ref[...])


_fin_call = pl.pallas_call(
    _fin_body,
    grid=(N // _B,),
    in_specs=[
        pl.BlockSpec((NC, _B, D), lambda i: (0, i, 0)),
        pl.BlockSpec((_B, D), lambda i: (i, 0)),
        pl.BlockSpec((_B, D), lambda i: (i, 0)),
        pl.BlockSpec((1, D), lambda i: (0, 0)),
    ],
    out_specs=pl.BlockSpec((_B, D), lambda i: (i, 0)),
    out_shape=jax.ShapeDtypeStruct((N, D), _f32),
)


def kernel(x, edge_index, W1, b1, W2, b2, W3, b3):
    ei = edge_index.astype(jnp.int32)
    pad = EPAD - E
    src = jnp.concatenate([ei[0], jnp.zeros((pad,), jnp.int32)])
    dst = jnp.concatenate([ei[1], jnp.full((pad,), N, jnp.int32)])
    srcr = src.reshape(NW, CPT, 1, CH)
    dstr = dst.reshape(NW, CPT, 1, CH)

    dst_deg = jnp.concatenate(
        [ei[1], jnp.full((EPAD_DEG - E,), N, jnp.int32)])
    degs = _deg_call()(dst_deg.reshape(NW, CPT_DEG, CH))
    dinv, hs = _pre_call(degs[:, :N], x, W1)

    agg = _agg_call()
    acc, _ = agg(hs, srcr, dstr)
    hs = _mid_call(acc[:, :N], hs[:N], dinv, b1.reshape(1, D), W2)
    acc, _ = agg(hs, srcr, dstr)
    hs = _mid_call(acc[:, :N], hs[:N], dinv, b2.reshape(1, D), W3)
    acc, _ = agg(hs, srcr, dstr)
    return _fin_call(acc[:, :N], hs[:N], dinv, b3.reshape(1, D))
